# Initial kernel scaffold; baseline (speedup 1.0000x reference)
import functools

import jax
import jax.numpy as jnp
from jax import lax
from jax.experimental import pallas as pl
from jax.experimental.pallas import tpu as pltpu
from jax.experimental.pallas import tpu_sc as plsc

N_NODES = 10000
N_EDGES = 320000
D = 128

NC = 2
NS = 16
NW = NC * NS

E_PER_W = N_EDGES // NW
K = 125
CHUNKS = E_PER_W // K
assert CHUNKS * K == E_PER_W

NPAD = 10016
ROWS_PER_S = NPAD // NS

XPAD = NW * 3 * 128


def _gather_body(ent_hbm, emb_hbm, x_hbm, idx_v, rows_v, sem):
    c = lax.axis_index("c")
    s = lax.axis_index("s")
    w = s * NC + c
    pltpu.sync_copy(ent_hbm.at[w], idx_v)
    for j in range(3):
        pltpu.async_copy(emb_hbm.at[idx_v.at[j]],
                         rows_v.at[pl.ds(j * 128, 128)], sem).wait()
    pltpu.sync_copy(rows_v, x_hbm.at[pl.ds(w * 384, 384)])


_gather_call = pl.kernel(
    _gather_body,
    out_type=jax.ShapeDtypeStruct((XPAD, D), jnp.float32),
    mesh=plsc.VectorSubcoreMesh(core_axis_name="c", subcore_axis_name="s"),
    scratch_types=[
        pltpu.VMEM((3, 128), jnp.int32),
        pltpu.VMEM((384, D), jnp.float32),
        pltpu.SemaphoreType.DMA,
    ],
)


def _agg_body(with_deg, src_hbm, dst_hbm, x_hbm, z128_hbm, z16_hbm, ones_hbm,
              agg_out, deg_out, src_v, dst_v, rows_v, ones_v, agg_sh, deg_sh,
              sem):
    c = lax.axis_index("c")
    s = lax.axis_index("s")
    w = s * NC + c

    pltpu.sync_copy(z128_hbm.at[pl.ds(s * ROWS_PER_S, ROWS_PER_S)],
                    agg_sh.at[pl.ds(s * ROWS_PER_S, ROWS_PER_S)])
    if with_deg:
        pltpu.sync_copy(z16_hbm.at[pl.ds(s * ROWS_PER_S, ROWS_PER_S)],
                        deg_sh.at[pl.ds(s * ROWS_PER_S, ROWS_PER_S)])
        pltpu.sync_copy(ones_hbm, ones_v)
    plsc.subcore_barrier()

    pltpu.sync_copy(src_hbm.at[w], src_v)
    pltpu.sync_copy(dst_hbm.at[w], dst_v)

    def body(j, carry):
        pltpu.async_copy(x_hbm.at[src_v.at[j]], rows_v, sem).wait()
        pltpu.sync_copy(rows_v, agg_sh.at[dst_v.at[j]], add=True)
        if with_deg:
            pltpu.sync_copy(ones_v, deg_sh.at[dst_v.at[j]], add=True)
        return carry

    lax.fori_loop(0, CHUNKS, body, 0)
    plsc.subcore_barrier()

    pltpu.sync_copy(agg_sh.at[pl.ds(s * ROWS_PER_S, ROWS_PER_S)],
                    agg_out.at[c, pl.ds(s * ROWS_PER_S, ROWS_PER_S)])
    if with_deg:
        pltpu.sync_copy(deg_sh.at[pl.ds(s * ROWS_PER_S, ROWS_PER_S)],
                        deg_out.at[c, pl.ds(s * ROWS_PER_S, ROWS_PER_S)])


def _make_agg_call(with_deg):
    return pl.kernel(
        functools.partial(_agg_body, with_deg),
        out_type=(
            jax.ShapeDtypeStruct((NC, NPAD, D), jnp.float32),
            jax.ShapeDtypeStruct((NC, NPAD, 16), jnp.float32),
        ),
        mesh=plsc.VectorSubcoreMesh(core_axis_name="c", subcore_axis_name="s"),
        scratch_types=[
            pltpu.VMEM((CHUNKS, K), jnp.int32),
            pltpu.VMEM((CHUNKS, K), jnp.int32),
            pltpu.VMEM((K, D), jnp.float32),
            pltpu.VMEM((K, 16), jnp.float32),
            pltpu.VMEM_SHARED((NPAD, D), jnp.float32),
            pltpu.VMEM_SHARED((NPAD, 16), jnp.float32),
            pltpu.SemaphoreType.DMA,
        ],
    )


_agg_call_deg = _make_agg_call(True)
_agg_call_nodeg = _make_agg_call(False)


def _mm_body(relu, agg_ref, deg_ref, x_ref, wl_ref, wr_ref, b_ref, out_ref):
    a = agg_ref[0] + agg_ref[1]
    dcol = deg_ref[0, :, 0:1] + deg_ref[1, :, 0:1]
    inv = 1.0 / jnp.maximum(dcol, 1.0)
    h = lax.dot_general(a * inv, wl_ref[...], (((1,), (1,)), ((), ())),
                        preferred_element_type=jnp.float32)
    h = h + b_ref[...]
    h = h + lax.dot_general(x_ref[...], wr_ref[...], (((1,), (1,)), ((), ())),
                            preferred_element_type=jnp.float32)
    if relu:
        h = jnp.maximum(h, 0.0)
    out_ref[...] = h


def _make_mm_call(relu):
    bm = 1000
    return pl.pallas_call(
        functools.partial(_mm_body, relu),
        grid=(N_NODES // bm,),
        in_specs=[
            pl.BlockSpec((NC, bm, D), lambda i: (0, i, 0)),
            pl.BlockSpec((NC, bm, 16), lambda i: (0, i, 0)),
            pl.BlockSpec((bm, D), lambda i: (i, 0)),
            pl.BlockSpec((D, D), lambda i: (0, 0)),
            pl.BlockSpec((D, D), lambda i: (0, 0)),
            pl.BlockSpec((1, D), lambda i: (0, 0)),
        ],
        out_specs=pl.BlockSpec((bm, D), lambda i: (i, 0)),
        out_shape=jax.ShapeDtypeStruct((N_NODES, D), jnp.float32),
    )


def kernel(entity, edge_index, edge_type, edge_norm, emb, rel,
           W1_l, b1_l, W1_r, W2_l, b2_l, W2_r):
    del edge_type, edge_norm, rel

    ent = jnp.concatenate(
        [entity.astype(jnp.int32),
         jnp.zeros((XPAD - N_NODES,), jnp.int32)]).reshape(NW, 3, 128)
    src = edge_index[0].astype(jnp.int32).reshape(NW, CHUNKS, K)
    dst = edge_index[1].astype(jnp.int32).reshape(NW, CHUNKS, K)
    z128 = jnp.zeros((NPAD, D), jnp.float32)
    z16 = jnp.zeros((NPAD, 16), jnp.float32)
    ones16 = jnp.ones((K, 16), jnp.float32)
    b1 = b1_l.reshape(1, D)
    b2 = b2_l.reshape(1, D)

    x = _gather_call(ent, emb)
    agg1, deg1 = _agg_call_deg(src, dst, x, z128, z16, ones16)
    h1 = _make_mm_call(True)(agg1, deg1, x[:N_NODES], W1_l, W1_r, b1)
    agg2, _ = _agg_call_nodeg(src, dst, h1, z128, z16, ones16)
    out = _make_mm_call(False)(agg2, deg1, h1, W2_l, W2_r, b2)
    return out


# speedup vs baseline: 5.9544x; 5.9544x over previous
import functools

import jax
import jax.numpy as jnp
from jax import lax
from jax.experimental import pallas as pl
from jax.experimental.pallas import tpu as pltpu
from jax.experimental.pallas import tpu_sc as plsc

N_NODES = 10000
N_EDGES = 320000
D = 128

NC = 2
NS = 16
NW = NC * NS

E_PER_W = N_EDGES // NW
K = 125
CHUNKS = E_PER_W // K
assert CHUNKS * K == E_PER_W

NPAD = 10112
ROWS_PER_S = NPAD // NS

XPAD = NW * 3 * 128


def _gather_body(ent_hbm, emb_hbm, x_hbm, idx_v, rows_v, sem):
    c = lax.axis_index("c")
    s = lax.axis_index("s")
    w = s * NC + c
    pltpu.sync_copy(ent_hbm.at[w], idx_v)
    for j in range(3):
        pltpu.async_copy(emb_hbm.at[idx_v.at[j]],
                         rows_v.at[pl.ds(j * 128, 128)], sem).wait()
    pltpu.sync_copy(rows_v, x_hbm.at[pl.ds(w * 384, 384)])


_gather_call = pl.kernel(
    _gather_body,
    out_type=jax.ShapeDtypeStruct((XPAD, D), jnp.float32),
    mesh=plsc.VectorSubcoreMesh(core_axis_name="c", subcore_axis_name="s"),
    scratch_types=[
        pltpu.VMEM((3, 128), jnp.int32),
        pltpu.VMEM((384, D), jnp.float32),
        pltpu.SemaphoreType.DMA,
    ],
)


def _deg_body(dst_hbm, ones_hbm, z128_hbm, deg_out, dst_v, ones_v, deg_sh):
    c = lax.axis_index("c")
    s = lax.axis_index("s")
    w = s * NC + c

    pltpu.sync_copy(z128_hbm.at[pl.ds(s * ROWS_PER_S, ROWS_PER_S)],
                    deg_sh.at[pl.ds(s * ROWS_PER_S, ROWS_PER_S)])
    pltpu.sync_copy(ones_hbm, ones_v)
    plsc.subcore_barrier()

    pltpu.sync_copy(dst_hbm.at[w], dst_v)

    def body(j, carry):
        pltpu.sync_copy(ones_v, deg_sh.at[dst_v.at[j]], add=True)
        return carry

    lax.fori_loop(0, CHUNKS, body, 0)
    plsc.subcore_barrier()

    pltpu.sync_copy(deg_sh.at[pl.ds(s * ROWS_PER_S, ROWS_PER_S)],
                    deg_out.at[c, pl.ds(s * ROWS_PER_S, ROWS_PER_S)])


_deg_call = pl.kernel(
    _deg_body,
    out_type=jax.ShapeDtypeStruct((NC, NPAD, D), jnp.float32),
    mesh=plsc.VectorSubcoreMesh(core_axis_name="c", subcore_axis_name="s"),
    scratch_types=[
        pltpu.VMEM((CHUNKS, K), jnp.int32),
        pltpu.VMEM((K, D), jnp.float32),
        pltpu.VMEM_SHARED((NPAD, D), jnp.float32),
    ],
)


def _agg_body(src_hbm, dst_hbm, x_hbm, z128_hbm, agg_out,
              src_v, dst_v, rows_v, agg_sh, sem):
    c = lax.axis_index("c")
    s = lax.axis_index("s")
    w = s * NC + c

    pltpu.sync_copy(z128_hbm.at[pl.ds(s * ROWS_PER_S, ROWS_PER_S)],
                    agg_sh.at[pl.ds(s * ROWS_PER_S, ROWS_PER_S)])
    plsc.subcore_barrier()

    pltpu.sync_copy(src_hbm.at[w], src_v)
    pltpu.sync_copy(dst_hbm.at[w], dst_v)

    def body(j, carry):
        pltpu.async_copy(x_hbm.at[src_v.at[j]], rows_v, sem).wait()
        pltpu.sync_copy(rows_v, agg_sh.at[dst_v.at[j]], add=True)
        return carry

    lax.fori_loop(0, CHUNKS, body, 0)
    plsc.subcore_barrier()

    pltpu.sync_copy(agg_sh.at[pl.ds(s * ROWS_PER_S, ROWS_PER_S)],
                    agg_out.at[c, pl.ds(s * ROWS_PER_S, ROWS_PER_S)])


_agg_call = pl.kernel(
    _agg_body,
    out_type=jax.ShapeDtypeStruct((NC, NPAD, D), jnp.float32),
    mesh=plsc.VectorSubcoreMesh(core_axis_name="c", subcore_axis_name="s"),
    scratch_types=[
        pltpu.VMEM((CHUNKS, K), jnp.int32),
        pltpu.VMEM((CHUNKS, K), jnp.int32),
        pltpu.VMEM((K, D), jnp.float32),
        pltpu.VMEM_SHARED((NPAD, D), jnp.float32),
        pltpu.SemaphoreType.DMA,
    ],
)


def _mm_body(relu, agg_ref, deg_ref, x_ref, wl_ref, wr_ref, b_ref, out_ref):
    a = agg_ref[0] + agg_ref[1]
    dcol = deg_ref[0, :, 0:1] + deg_ref[1, :, 0:1]
    inv = 1.0 / jnp.maximum(dcol, 1.0)
    h = lax.dot_general(a * inv, wl_ref[...], (((1,), (1,)), ((), ())),
                        preferred_element_type=jnp.float32)
    h = h + b_ref[...]
    h = h + lax.dot_general(x_ref[...], wr_ref[...], (((1,), (1,)), ((), ())),
                            preferred_element_type=jnp.float32)
    if relu:
        h = jnp.maximum(h, 0.0)
    out_ref[...] = h


def _make_mm_call(relu):
    bm = 1000
    return pl.pallas_call(
        functools.partial(_mm_body, relu),
        grid=(N_NODES // bm,),
        in_specs=[
            pl.BlockSpec((NC, bm, D), lambda i: (0, i, 0)),
            pl.BlockSpec((NC, bm, D), lambda i: (0, i, 0)),
            pl.BlockSpec((bm, D), lambda i: (i, 0)),
            pl.BlockSpec((D, D), lambda i: (0, 0)),
            pl.BlockSpec((D, D), lambda i: (0, 0)),
            pl.BlockSpec((1, D), lambda i: (0, 0)),
        ],
        out_specs=pl.BlockSpec((bm, D), lambda i: (i, 0)),
        out_shape=jax.ShapeDtypeStruct((N_NODES, D), jnp.float32),
    )


def kernel(entity, edge_index, edge_type, edge_norm, emb, rel,
           W1_l, b1_l, W1_r, W2_l, b2_l, W2_r):
    del edge_type, edge_norm, rel

    ent = jnp.concatenate(
        [entity.astype(jnp.int32),
         jnp.zeros((XPAD - N_NODES,), jnp.int32)]).reshape(NW, 3, 128)
    src = edge_index[0].astype(jnp.int32).reshape(NW, CHUNKS, K)
    dst = edge_index[1].astype(jnp.int32).reshape(NW, CHUNKS, K)
    z128 = jnp.zeros((NPAD, D), jnp.float32)
    ones128 = jnp.ones((K, D), jnp.float32)
    b1 = b1_l.reshape(1, D)
    b2 = b2_l.reshape(1, D)

    x = _gather_call(ent, emb)
    deg1 = _deg_call(dst, ones128, z128)
    agg1 = _agg_call(src, dst, x, z128)
    h1 = _make_mm_call(True)(agg1, deg1, x, W1_l, W1_r, b1)
    agg2 = _agg_call(src, dst, h1, z128)
    out = _make_mm_call(False)(agg2, deg1, h1, W2_l, W2_r, b2)
    return out
